# baseline (speedup 1.0000x reference)
import functools
import jax
import jax.numpy as jnp
from jax import lax
from jax.experimental import pallas as pl
from jax.experimental.pallas import tpu as pltpu
from jax.experimental.pallas import tpu_sc as plsc

N = 10000
E = 320000
D_IN = 128
H2 = 256
HID = 128
PROJ = 64

NC = 2
NS = 16
CH = 128
NB = 4
CPT_F = 160
CPT_E = 80
E_PAD = NC * NS * CPT_E * CH
PADC = E_PAD // CH
NPAD = 10240
RPT = NPAD // NS
DUMMY = N

RB = 400
GRID = N // RB

_mesh = plsc.VectorSubcoreMesh(
    core_axis_name="c", subcore_axis_name="s", num_cores=NC, num_subcores=NS
)


def _deg_body(sd_hbm, z_hbm, deg_hbm, di, ones_b, acc, *sems):
    c = lax.axis_index("c")
    s = lax.axis_index("s")

    @pl.when(c == 0)
    def _():
        for k in range(CH // 16):
            ones_b[pl.ds(k * 16, 16)] = jnp.ones((16,), jnp.float32)
        pltpu.sync_copy(z_hbm.at[pl.ds(s * RPT, RPT)], acc.at[pl.ds(s * RPT, RPT)])
        pltpu.sync_copy(sd_hbm.at[pl.ds(s * CPT_F, CPT_F)], di)
        plsc.subcore_barrier()

        def scat(j, b):
            return pltpu.make_async_copy(ones_b, acc.at[di.at[j, 1]], sems[b])

        def grp(g, carry):
            for b in range(NB):
                j = g * NB + b
                scat(j, b).start(add=True)
            for b in range(NB):
                scat(0, b).wait()
            return carry

        lax.fori_loop(0, CPT_F // NB, grp, 0)
        plsc.subcore_barrier()
        pltpu.sync_copy(acc.at[pl.ds(s * RPT, RPT)], deg_hbm.at[pl.ds(s * RPT, RPT)])


def _deg_call(sd_pad, z1):
    f = pl.kernel(
        _deg_body,
        out_type=jax.ShapeDtypeStruct((NPAD,), jnp.float32),
        mesh=_mesh,
        scratch_types=[
            pltpu.VMEM((CPT_F, 2, CH), jnp.int32),
            pltpu.VMEM((CH,), jnp.float32),
            pltpu.VMEM_SHARED((NPAD,), jnp.float32),
        ]
        + [pltpu.SemaphoreType.DMA] * NB,
    )
    return f(sd_pad, z1)


def _mp_common(cpt, chunk_base, y_hbm, sd_hbm, sdb, rows, acc, sems, stride=1):
    gs, ss, ix = sems[0:2], sems[2:4], sems[4:6]

    def idx(j, b):
        return pltpu.make_async_copy(sd_hbm.at[chunk_base + stride * j], sdb[b], ix[b])

    def gat(j, b):
        return pltpu.make_async_copy(y_hbm.at[sdb[b].at[0]], rows[b], gs[b])

    def scat(b):
        return pltpu.make_async_copy(rows[b], acc.at[sdb[b].at[1]], ss[b])

    plsc.subcore_barrier()
    idx(0, 0).start()
    idx(1, 1).start()
    idx(0, 0).wait()
    gat(0, 0).start()

    def grp(g, carry):
        for b in range(2):
            j = 2 * g + b
            gat(j, b).wait()
            scat(b).start(add=True)

            @pl.when(j + 2 < cpt)
            def _():
                idx(j + 2, b).start()

            @pl.when(j + 1 < cpt)
            def _():
                @pl.when(j >= 1)
                def _():
                    scat(1 - b).wait()

                idx(0, 1 - b).wait()
                gat(j + 1, 1 - b).start()

        return carry

    lax.fori_loop(0, cpt // 2, grp, 0)
    scat(0).wait()
    scat(1).wait()
    plsc.subcore_barrier()


def _mp_writeback(c, s, acc, s0_out, s1_out):
    @pl.when(c == 0)
    def _():
        pltpu.sync_copy(acc.at[pl.ds(s * RPT, RPT)], s0_out.at[pl.ds(s * RPT, RPT)])

    @pl.when(c == 1)
    def _():
        pltpu.sync_copy(acc.at[pl.ds(s * RPT, RPT)], s1_out.at[pl.ds(s * RPT, RPT)])


def _mp_body(hc, y0, y1, sd_hbm, z_hbm, s0_out, s1_out,
             sd0, sd1, r0, r1, acc, *sems):
    c = lax.axis_index("c")
    s = lax.axis_index("s")

    pltpu.sync_copy(z_hbm.at[pl.ds(s * RPT, RPT)], acc.at[pl.ds(s * RPT, RPT)])

    @pl.when(c == 0)
    def _():
        _mp_common(CPT_F, s * CPT_F, y0, sd_hbm, (sd0, sd1), (r0, r1), acc, sems)

    @pl.when(c == 1)
    def _():
        _mp_common(CPT_F, s * CPT_F, y1, sd_hbm, (sd0, sd1), (r0, r1), acc, sems)

    _mp_writeback(c, s, acc, s0_out, s1_out)


def _mp_call(hc, y0, y1, sd_pad, z2):
    f = pl.kernel(
        functools.partial(_mp_body, hc),
        out_type=[jax.ShapeDtypeStruct((NPAD, hc), jnp.float32)] * 2,
        mesh=_mesh,
        scratch_types=[
            pltpu.VMEM((2, CH), jnp.int32),
            pltpu.VMEM((2, CH), jnp.int32),
            pltpu.VMEM((CH, hc), jnp.float32),
            pltpu.VMEM((CH, hc), jnp.float32),
            pltpu.VMEM_SHARED((NPAD, hc), jnp.float32),
        ]
        + [pltpu.SemaphoreType.DMA] * 6,
    )
    return f(y0, y1, sd_pad, z2)


def _mp_edge_body(hc, y0, y1, sd_hbm, z_hbm, s0_out, s1_out,
                  sd0, sd1, r0, r1, acc, *sems):
    c = lax.axis_index("c")
    s = lax.axis_index("s")

    pltpu.sync_copy(z_hbm.at[pl.ds(s * RPT, RPT)], acc.at[pl.ds(s * RPT, RPT)])

    @pl.when(c == 0)
    def _():
        _mp_common(CPT_E, NC * s * CPT_E, y0, sd_hbm, (sd0, sd1), (r0, r1),
                   acc, sems, stride=NC)

    @pl.when(c == 1)
    def _():
        _mp_common(CPT_E, NC * s * CPT_E + 1, y1, sd_hbm, (sd0, sd1), (r0, r1),
                   acc, sems, stride=NC)

    _mp_writeback(c, s, acc, s0_out, s1_out)


def _mp_edge_call(hc, y0, y1, sd_pad, z):
    f = pl.kernel(
        functools.partial(_mp_edge_body, hc),
        out_type=[jax.ShapeDtypeStruct((NPAD, hc), jnp.float32)] * 2,
        mesh=_mesh,
        scratch_types=[
            pltpu.VMEM((2, CH), jnp.int32),
            pltpu.VMEM((2, CH), jnp.int32),
            pltpu.VMEM((CH, hc), jnp.float32),
            pltpu.VMEM((CH, hc), jnp.float32),
            pltpu.VMEM_SHARED((NPAD, hc), jnp.float32),
        ]
        + [pltpu.SemaphoreType.DMA] * 6,
    )
    return f(y0, y1, sd_pad, z)


def _dis(deg_ref):
    return lax.rsqrt(deg_ref[...] + 1.0)


def _lin1_body(x_ref, w_ref, deg_ref, y0_ref, y1_ref):
    dis = _dis(deg_ref)
    y = jnp.dot(x_ref[...], w_ref[...], preferred_element_type=jnp.float32) * dis
    y0_ref[...] = y[:, : H2 // 2]
    y1_ref[...] = y[:, H2 // 2 :]


def _lin1_call(x, w0, deg2):
    return pl.pallas_call(
        _lin1_body,
        grid=(GRID,),
        in_specs=[
            pl.BlockSpec((RB, D_IN), lambda i: (i, 0)),
            pl.BlockSpec((D_IN, H2), lambda i: (0, 0)),
            pl.BlockSpec((RB, 1), lambda i: (i, 0)),
        ],
        out_specs=[
            pl.BlockSpec((RB, H2 // 2), lambda i: (i, 0)),
            pl.BlockSpec((RB, H2 // 2), lambda i: (i, 0)),
        ],
        out_shape=[jax.ShapeDtypeStruct((N, H2 // 2), jnp.float32)] * 2,
    )(x, w0, deg2)


def _mid_body(split_out, s0_ref, s1_ref, y0_ref, y1_ref, deg_ref, w_ref, b_ref,
              *out_refs):
    dis = _dis(deg_ref)
    h = jnp.concatenate(
        [s0_ref[...] + y0_ref[...], s1_ref[...] + y1_ref[...]], axis=1
    )
    h = jax.nn.relu(dis * h + b_ref[...])
    y = jnp.dot(h, w_ref[...], preferred_element_type=jnp.float32) * dis
    if split_out:
        hh = w_ref.shape[1] // 2
        out_refs[0][...] = y[:, :hh]
        out_refs[1][...] = y[:, hh:]
    else:
        out_refs[0][...] = y
        out_refs[1][...] = y


def _mid_call(s0, s1, y0, y1, deg2, w, b2d, split_out=True):
    hin = w.shape[0]
    hout = w.shape[1]
    if split_out:
        out_specs = [
            pl.BlockSpec((RB, hout // 2), lambda i: (i, 0)),
            pl.BlockSpec((RB, hout // 2), lambda i: (i, 0)),
        ]
        out_shape = [jax.ShapeDtypeStruct((N, hout // 2), jnp.float32)] * 2
    else:
        out_specs = [
            pl.BlockSpec((RB, hout), lambda i: (i, 0)),
            pl.BlockSpec((RB, hout), lambda i: (i, 0)),
        ]
        out_shape = [jax.ShapeDtypeStruct((N, hout), jnp.float32)] * 2
    return pl.pallas_call(
        functools.partial(_mid_body, split_out),
        grid=(GRID,),
        in_specs=[
            pl.BlockSpec((RB, hin // 2), lambda i: (i, 0)),
            pl.BlockSpec((RB, hin // 2), lambda i: (i, 0)),
            pl.BlockSpec((RB, hin // 2), lambda i: (i, 0)),
            pl.BlockSpec((RB, hin // 2), lambda i: (i, 0)),
            pl.BlockSpec((RB, 1), lambda i: (i, 0)),
            pl.BlockSpec((hin, hout), lambda i: (0, 0)),
            pl.BlockSpec((1, hin), lambda i: (0, 0)),
        ],
        out_specs=out_specs,
        out_shape=out_shape,
    )(s0, s1, y0, y1, deg2, w, b2d)


def _fin_body(s0_ref, s1_ref, y_ref, deg_ref, b2_ref, wp_ref, bp_ref, out_ref):
    dis = _dis(deg_ref)
    h = s0_ref[...] + s1_ref[...] + y_ref[...]
    h = jax.nn.relu(dis * h + b2_ref[...])
    p = jax.nn.relu(
        jnp.dot(h, wp_ref[...], preferred_element_type=jnp.float32) + bp_ref[...]
    )
    nrm = jnp.sqrt(jnp.sum(p * p, axis=1, keepdims=True))
    out_ref[...] = p / jnp.maximum(nrm, 1e-12)


def _fin_call(s0, s1, y, deg2, b2d, wp, bp2d):
    return pl.pallas_call(
        _fin_body,
        grid=(GRID,),
        in_specs=[
            pl.BlockSpec((RB, HID), lambda i: (i, 0)),
            pl.BlockSpec((RB, HID), lambda i: (i, 0)),
            pl.BlockSpec((RB, HID), lambda i: (i, 0)),
            pl.BlockSpec((RB, 1), lambda i: (i, 0)),
            pl.BlockSpec((1, HID), lambda i: (0, 0)),
            pl.BlockSpec((HID, PROJ), lambda i: (0, 0)),
            pl.BlockSpec((1, PROJ), lambda i: (0, 0)),
        ],
        out_specs=pl.BlockSpec((RB, PROJ), lambda i: (i, 0)),
        out_shape=jax.ShapeDtypeStruct((N, PROJ), jnp.float32),
    )(s0, s1, y, deg2, b2d, wp, bp2d)


def kernel(x, edge_index1, W0, b0, W1, b1, W2, b2, Wp, bp):
    pad = E_PAD - E
    src_pad = jnp.concatenate([edge_index1[0], jnp.zeros((pad,), jnp.int32)])
    dst_pad = jnp.concatenate(
        [edge_index1[1], DUMMY + (jnp.arange(pad, dtype=jnp.int32) % (NPAD - N))]
    )
    sd_pad = jnp.stack(
        [src_pad.reshape(PADC, CH), dst_pad.reshape(PADC, CH)], axis=1
    )
    z1 = jnp.zeros((NPAD,), jnp.float32)
    z2 = jnp.zeros((NPAD, H2 // 2), jnp.float32)

    deg = _deg_call(sd_pad, z1)
    deg2 = deg[:, None]

    y0a, y0b = _lin1_call(x, W0, deg2)
    s1a, s1b = _mp_call(H2 // 2, y0a, y0b, sd_pad, z2)

    y1a, y1b = _mid_call(s1a, s1b, y0a, y0b, deg2, W1, b0[None, :])
    s2a, s2b = _mp_call(H2 // 2, y1a, y1b, sd_pad, z2)

    y2a, y2b = _mid_call(s2a, s2b, y1a, y1b, deg2, W2, b1[None, :],
                         split_out=False)
    s3a, s3b = _mp_edge_call(HID, y2a, y2b, sd_pad, z2)

    return _fin_call(s3a, s3b, y2a, deg2, b2[None, :], Wp, bp[None, :])


# speedup vs baseline: 2.4202x; 2.4202x over previous
import functools
import jax
import jax.numpy as jnp
from jax import lax
from jax.experimental import pallas as pl
from jax.experimental.pallas import tpu as pltpu
from jax.experimental.pallas import tpu_sc as plsc

N = 10000
E = 320000
D_IN = 128
H2 = 256
HID = 128
PROJ = 64

NC = 2
NS = 16
CH = 128
NB = 4
CPT_F = 160
CPT_E = 80
E_PAD = NC * NS * CPT_E * CH
PADC = E_PAD // CH
NPAD = 10240
RPT = NPAD // NS
DUMMY = N

RB = 400
GRID = N // RB
RB2 = 512
GRID2 = NPAD // RB2

_mesh = plsc.VectorSubcoreMesh(
    core_axis_name="c", subcore_axis_name="s", num_cores=NC, num_subcores=NS
)


def _deg_body(sd_hbm, z_hbm, deg_hbm, di, ones_b, acc, *sems):
    c = lax.axis_index("c")
    s = lax.axis_index("s")

    @pl.when(c == 0)
    def _():
        for k in range(CH // 16):
            ones_b[pl.ds(k * 16, 16)] = jnp.ones((16,), jnp.float32)
        pltpu.sync_copy(z_hbm.at[pl.ds(s * RPT, RPT)], acc.at[pl.ds(s * RPT, RPT)])
        pltpu.sync_copy(sd_hbm.at[pl.ds(s * CPT_F, CPT_F)], di)
        plsc.subcore_barrier()

        def scat(j, b):
            return pltpu.make_async_copy(ones_b, acc.at[di.at[j, 0]], sems[b])

        def grp(g, carry):
            for b in range(NB):
                j = g * NB + b
                scat(j, b).start(add=True)
            for b in range(NB):
                scat(0, b).wait()
            return carry

        lax.fori_loop(0, CPT_F // NB, grp, 0)
        plsc.subcore_barrier()
        pltpu.sync_copy(acc.at[pl.ds(s * RPT, RPT)], deg_hbm.at[pl.ds(s * RPT, RPT)])


def _deg_call(sd_pad, z1):
    f = pl.kernel(
        _deg_body,
        out_type=jax.ShapeDtypeStruct((NPAD,), jnp.float32),
        mesh=_mesh,
        scratch_types=[
            pltpu.VMEM((CPT_F, 1, CH), jnp.int32),
            pltpu.VMEM((CH,), jnp.float32),
            pltpu.VMEM_SHARED((NPAD,), jnp.float32),
        ]
        + [pltpu.SemaphoreType.DMA] * NB,
    )
    return f(sd_pad, z1)


def _mp_common(cpt, chunk_base, y_hbm, sd_hbm, sdb, rows, acc, sems, stride=1):
    gs, ss, ix = sems[0:2], sems[2:4], sems[4:6]

    def idx(j, b):
        return pltpu.make_async_copy(sd_hbm.at[chunk_base + stride * j], sdb[b], ix[b])

    def gat(j, b):
        return pltpu.make_async_copy(y_hbm.at[sdb[b].at[0]], rows[b], gs[b])

    def scat(b):
        return pltpu.make_async_copy(rows[b], acc.at[sdb[b].at[1]], ss[b])

    plsc.subcore_barrier()
    idx(0, 0).start()
    idx(1, 1).start()
    idx(0, 0).wait()
    gat(0, 0).start()

    def grp(g, carry):
        for b in range(2):
            j = 2 * g + b
            gat(j, b).wait()
            scat(b).start(add=True)

            @pl.when(j + 2 < cpt)
            def _():
                idx(j + 2, b).start()

            @pl.when(j + 1 < cpt)
            def _():
                @pl.when(j >= 1)
                def _():
                    scat(1 - b).wait()

                idx(0, 1 - b).wait()
                gat(j + 1, 1 - b).start()

        return carry

    lax.fori_loop(0, cpt // 2, grp, 0)
    scat(0).wait()
    scat(1).wait()
    plsc.subcore_barrier()


def _mp_writeback(c, s, acc, s0_out, s1_out):
    @pl.when(c == 0)
    def _():
        pltpu.sync_copy(acc.at[pl.ds(s * RPT, RPT)], s0_out.at[pl.ds(s * RPT, RPT)])

    @pl.when(c == 1)
    def _():
        pltpu.sync_copy(acc.at[pl.ds(s * RPT, RPT)], s1_out.at[pl.ds(s * RPT, RPT)])


def _mp_body(hc, y0, y1, sd_hbm, z_hbm, s0_out, s1_out,
             sd0, sd1, r0, r1, acc, *sems):
    c = lax.axis_index("c")
    s = lax.axis_index("s")

    pltpu.sync_copy(z_hbm.at[pl.ds(s * RPT, RPT)], acc.at[pl.ds(s * RPT, RPT)])

    @pl.when(c == 0)
    def _():
        _mp_common(CPT_F, s * CPT_F, y0, sd_hbm, (sd0, sd1), (r0, r1), acc, sems)

    @pl.when(c == 1)
    def _():
        _mp_common(CPT_F, s * CPT_F, y1, sd_hbm, (sd0, sd1), (r0, r1), acc, sems)

    _mp_writeback(c, s, acc, s0_out, s1_out)


def _mp_call(hc, y0, y1, sd_pad, z2):
    f = pl.kernel(
        functools.partial(_mp_body, hc),
        out_type=[jax.ShapeDtypeStruct((NPAD, hc), jnp.float32)] * 2,
        mesh=_mesh,
        scratch_types=[
            pltpu.VMEM((2, CH), jnp.int32),
            pltpu.VMEM((2, CH), jnp.int32),
            pltpu.VMEM((CH, hc), jnp.float32),
            pltpu.VMEM((CH, hc), jnp.float32),
            pltpu.VMEM_SHARED((NPAD, hc), jnp.float32),
        ]
        + [pltpu.SemaphoreType.DMA] * 6,
    )
    return f(y0, y1, sd_pad, z2)


def _mp_edge_body(hc, y0, y1, sd_hbm, z_hbm, s0_out, s1_out,
                  sd0, sd1, r0, r1, acc, *sems):
    c = lax.axis_index("c")
    s = lax.axis_index("s")

    pltpu.sync_copy(z_hbm.at[pl.ds(s * RPT, RPT)], acc.at[pl.ds(s * RPT, RPT)])

    @pl.when(c == 0)
    def _():
        _mp_common(CPT_E, NC * s * CPT_E, y0, sd_hbm, (sd0, sd1), (r0, r1),
                   acc, sems, stride=NC)

    @pl.when(c == 1)
    def _():
        _mp_common(CPT_E, NC * s * CPT_E + 1, y1, sd_hbm, (sd0, sd1), (r0, r1),
                   acc, sems, stride=NC)

    _mp_writeback(c, s, acc, s0_out, s1_out)


def _mp_edge_call(hc, y0, y1, sd_pad, z):
    f = pl.kernel(
        functools.partial(_mp_edge_body, hc),
        out_type=[jax.ShapeDtypeStruct((NPAD, hc), jnp.float32)] * 2,
        mesh=_mesh,
        scratch_types=[
            pltpu.VMEM((2, CH), jnp.int32),
            pltpu.VMEM((2, CH), jnp.int32),
            pltpu.VMEM((CH, hc), jnp.float32),
            pltpu.VMEM((CH, hc), jnp.float32),
            pltpu.VMEM_SHARED((NPAD, hc), jnp.float32),
        ]
        + [pltpu.SemaphoreType.DMA] * 6,
    )
    return f(y0, y1, sd_pad, z)


def _dis(deg_ref):
    return lax.rsqrt(deg_ref[...] + 1.0)


def _lin1_body(x_ref, w_ref, deg_ref, y0_ref, y1_ref):
    dis = _dis(deg_ref)
    y = jnp.dot(x_ref[...], w_ref[...], preferred_element_type=jnp.float32) * dis
    y0_ref[...] = y[:, : H2 // 2]
    y1_ref[...] = y[:, H2 // 2 :]


def _lin1_call(x_pad, w0, deg2):
    return pl.pallas_call(
        _lin1_body,
        grid=(GRID2,),
        in_specs=[
            pl.BlockSpec((RB2, D_IN), lambda i: (i, 0)),
            pl.BlockSpec((D_IN, H2), lambda i: (0, 0)),
            pl.BlockSpec((RB2, 1), lambda i: (i, 0)),
        ],
        out_specs=[
            pl.BlockSpec((RB2, H2 // 2), lambda i: (i, 0)),
            pl.BlockSpec((RB2, H2 // 2), lambda i: (i, 0)),
        ],
        out_shape=[jax.ShapeDtypeStruct((NPAD, H2 // 2), jnp.float32)] * 2,
    )(x_pad, w0, deg2)


def _mid_body(split_out, s0_ref, s1_ref, y0_ref, y1_ref, deg_ref, w_ref, b_ref,
              *out_refs):
    dis = _dis(deg_ref)
    h = jnp.concatenate(
        [s0_ref[...] + y0_ref[...], s1_ref[...] + y1_ref[...]], axis=1
    )
    h = jax.nn.relu(dis * h + b_ref[...])
    y = jnp.dot(h, w_ref[...], preferred_element_type=jnp.float32) * dis
    row = pl.program_id(0) * RB2 + jax.lax.broadcasted_iota(
        jnp.int32, (RB2, 1), 0
    )
    y = jnp.where(row < N, y, 0.0)
    if split_out:
        hh = w_ref.shape[1] // 2
        out_refs[0][...] = y[:, :hh]
        out_refs[1][...] = y[:, hh:]
    else:
        out_refs[0][...] = y
        out_refs[1][...] = y


def _mid_call(s0, s1, y0, y1, deg2, w, b2d, split_out=True):
    hin = w.shape[0]
    hout = w.shape[1]
    if split_out:
        out_specs = [
            pl.BlockSpec((RB2, hout // 2), lambda i: (i, 0)),
            pl.BlockSpec((RB2, hout // 2), lambda i: (i, 0)),
        ]
        out_shape = [jax.ShapeDtypeStruct((NPAD, hout // 2), jnp.float32)] * 2
    else:
        out_specs = [
            pl.BlockSpec((RB2, hout), lambda i: (i, 0)),
            pl.BlockSpec((RB2, hout), lambda i: (i, 0)),
        ]
        out_shape = [jax.ShapeDtypeStruct((NPAD, hout), jnp.float32)] * 2
    return pl.pallas_call(
        functools.partial(_mid_body, split_out),
        grid=(GRID2,),
        in_specs=[
            pl.BlockSpec((RB2, hin // 2), lambda i: (i, 0)),
            pl.BlockSpec((RB2, hin // 2), lambda i: (i, 0)),
            pl.BlockSpec((RB2, hin // 2), lambda i: (i, 0)),
            pl.BlockSpec((RB2, hin // 2), lambda i: (i, 0)),
            pl.BlockSpec((RB2, 1), lambda i: (i, 0)),
            pl.BlockSpec((hin, hout), lambda i: (0, 0)),
            pl.BlockSpec((1, hin), lambda i: (0, 0)),
        ],
        out_specs=out_specs,
        out_shape=out_shape,
    )(s0, s1, y0, y1, deg2, w, b2d)


def _fin_body(s0_ref, s1_ref, y_ref, deg_ref, b2_ref, wp_ref, bp_ref, out_ref):
    dis = _dis(deg_ref)
    h = s0_ref[...] + s1_ref[...] + y_ref[...]
    h = jax.nn.relu(dis * h + b2_ref[...])
    p = jax.nn.relu(
        jnp.dot(h, wp_ref[...], preferred_element_type=jnp.float32) + bp_ref[...]
    )
    nrm = jnp.sqrt(jnp.sum(p * p, axis=1, keepdims=True))
    out_ref[...] = p / jnp.maximum(nrm, 1e-12)


def _fin_call(s0, s1, y, deg2, b2d, wp, bp2d):
    return pl.pallas_call(
        _fin_body,
        grid=(GRID,),
        in_specs=[
            pl.BlockSpec((RB, HID), lambda i: (i, 0)),
            pl.BlockSpec((RB, HID), lambda i: (i, 0)),
            pl.BlockSpec((RB, HID), lambda i: (i, 0)),
            pl.BlockSpec((RB, 1), lambda i: (i, 0)),
            pl.BlockSpec((1, HID), lambda i: (0, 0)),
            pl.BlockSpec((HID, PROJ), lambda i: (0, 0)),
            pl.BlockSpec((1, PROJ), lambda i: (0, 0)),
        ],
        out_specs=pl.BlockSpec((RB, PROJ), lambda i: (i, 0)),
        out_shape=jax.ShapeDtypeStruct((N, PROJ), jnp.float32),
    )(s0, s1, y, deg2, b2d, wp, bp2d)


def kernel(x, edge_index1, W0, b0, W1, b1, W2, b2, Wp, bp):
    pad = E_PAD - E
    ar = jnp.arange(pad, dtype=jnp.int32)
    src_pad = jnp.concatenate([edge_index1[0], DUMMY + ar % (NPAD - N)])
    dst_pad = jnp.concatenate([edge_index1[1], ar % NPAD])
    sd_pad = jnp.stack(
        [src_pad.reshape(PADC, CH), dst_pad.reshape(PADC, CH)], axis=1
    )
    dd_pad = jnp.concatenate(
        [edge_index1[1], DUMMY + ar % (NPAD - N)]
    ).reshape(PADC, 1, CH)
    z1 = jnp.zeros((NPAD,), jnp.float32)
    z2 = jnp.zeros((NPAD, H2 // 2), jnp.float32)
    x_pad = jnp.concatenate([x, jnp.zeros((NPAD - N, D_IN), jnp.float32)])

    deg = _deg_call(dd_pad, z1)
    deg2 = deg[:, None]

    y0a, y0b = _lin1_call(x_pad, W0, deg2)
    s1a, s1b = _mp_call(H2 // 2, y0a, y0b, sd_pad, z2)

    y1a, y1b = _mid_call(s1a, s1b, y0a, y0b, deg2, W1, b0[None, :])
    s2a, s2b = _mp_call(H2 // 2, y1a, y1b, sd_pad, z2)

    y2a, y2b = _mid_call(s2a, s2b, y1a, y1b, deg2, W2, b1[None, :],
                         split_out=False)
    s3a, s3b = _mp_edge_call(HID, y2a, y2b, sd_pad, z2)

    return _fin_call(s3a, s3b, y2a, deg2, b2[None, :], Wp, bp[None, :])


# speedup vs baseline: 2.4354x; 1.0063x over previous
import functools
import jax
import jax.numpy as jnp
from jax import lax
from jax.experimental import pallas as pl
from jax.experimental.pallas import tpu as pltpu
from jax.experimental.pallas import tpu_sc as plsc

N = 10000
E = 320000
D_IN = 128
H2 = 256
HID = 128
PROJ = 64

NC = 2
NS = 16
CH = 128
NB = 4
CPT_F = 160
CPT_E = 80
E_PAD = NC * NS * CPT_E * CH
PADC = E_PAD // CH
NPAD = 10240
RPT = NPAD // NS
DUMMY = N

RB = 400
GRID = N // RB
RB2 = 512
GRID2 = NPAD // RB2

_mesh = plsc.VectorSubcoreMesh(
    core_axis_name="c", subcore_axis_name="s", num_cores=NC, num_subcores=NS
)


def _deg_body(sd_hbm, z_hbm, deg_hbm, di, ones_b, acc, *sems):
    c = lax.axis_index("c")
    s = lax.axis_index("s")

    @pl.when(c == 0)
    def _():
        for k in range(CH // 16):
            ones_b[pl.ds(k * 16, 16)] = jnp.ones((16,), jnp.float32)
        pltpu.sync_copy(z_hbm.at[pl.ds(s * RPT, RPT)], acc.at[pl.ds(s * RPT, RPT)])
        pltpu.sync_copy(sd_hbm.at[pl.ds(s * CPT_F, CPT_F)], di)
        plsc.subcore_barrier()

        def scat(j, b):
            return pltpu.make_async_copy(ones_b, acc.at[di.at[j, 0]], sems[b])

        def grp(g, carry):
            for b in range(NB):
                j = g * NB + b
                scat(j, b).start(add=True)
            for b in range(NB):
                scat(0, b).wait()
            return carry

        lax.fori_loop(0, CPT_F // NB, grp, 0)
        plsc.subcore_barrier()
        pltpu.sync_copy(acc.at[pl.ds(s * RPT, RPT)], deg_hbm.at[pl.ds(s * RPT, RPT)])


def _deg_call(sd_pad, z1):
    f = pl.kernel(
        _deg_body,
        out_type=jax.ShapeDtypeStruct((NPAD,), jnp.float32),
        mesh=_mesh,
        scratch_types=[
            pltpu.VMEM((CPT_F, 1, CH), jnp.int32),
            pltpu.VMEM((CH,), jnp.float32),
            pltpu.VMEM_SHARED((NPAD,), jnp.float32),
        ]
        + [pltpu.SemaphoreType.DMA] * NB,
    )
    return f(sd_pad, z1)


def _mp_common(cpt, chunk_base, y_hbm, sd_hbm, sdb, rows, acc, sems, stride=1):
    gs, ss, ix = sems[0:2], sems[2:4], sems[4:6]

    def idx(j, b):
        return pltpu.make_async_copy(sd_hbm.at[chunk_base + stride * j], sdb[b], ix[b])

    def gat(j, b):
        return pltpu.make_async_copy(y_hbm.at[sdb[b].at[0]], rows[b], gs[b])

    def scat(b):
        return pltpu.make_async_copy(rows[b], acc.at[sdb[b].at[1]], ss[b])

    plsc.subcore_barrier()
    idx(0, 0).start()
    idx(1, 1).start()
    idx(0, 0).wait()
    gat(0, 0).start()

    def grp(g, carry):
        for b in range(2):
            j = 2 * g + b
            gat(j, b).wait()
            scat(b).start(add=True)

            @pl.when(j + 2 < cpt)
            def _():
                idx(j + 2, b).start()

            @pl.when(j + 1 < cpt)
            def _():
                @pl.when(j >= 1)
                def _():
                    scat(1 - b).wait()

                idx(0, 1 - b).wait()
                gat(j + 1, 1 - b).start()

        return carry

    lax.fori_loop(0, cpt // 2, grp, 0)
    scat(0).wait()
    scat(1).wait()
    plsc.subcore_barrier()


def _mp_writeback(c, s, acc, s0_out, s1_out):
    @pl.when(c == 0)
    def _():
        pltpu.sync_copy(acc.at[pl.ds(s * RPT, RPT)], s0_out.at[pl.ds(s * RPT, RPT)])

    @pl.when(c == 1)
    def _():
        pltpu.sync_copy(acc.at[pl.ds(s * RPT, RPT)], s1_out.at[pl.ds(s * RPT, RPT)])


def _mp_body(hc, y0, y1, sd_hbm, s0_out, s1_out,
             sd0, sd1, r0, r1, acc, *sems):
    c = lax.axis_index("c")
    s = lax.axis_index("s")

    @pl.when(c == 0)
    def _():
        pltpu.sync_copy(y0.at[pl.ds(s * RPT, RPT)], acc.at[pl.ds(s * RPT, RPT)])
        _mp_common(CPT_F, s * CPT_F, y0, sd_hbm, (sd0, sd1), (r0, r1), acc, sems)

    @pl.when(c == 1)
    def _():
        pltpu.sync_copy(y1.at[pl.ds(s * RPT, RPT)], acc.at[pl.ds(s * RPT, RPT)])
        _mp_common(CPT_F, s * CPT_F, y1, sd_hbm, (sd0, sd1), (r0, r1), acc, sems)

    _mp_writeback(c, s, acc, s0_out, s1_out)


def _mp_call(hc, y0, y1, sd_pad):
    f = pl.kernel(
        functools.partial(_mp_body, hc),
        out_type=[jax.ShapeDtypeStruct((NPAD, hc), jnp.float32)] * 2,
        mesh=_mesh,
        scratch_types=[
            pltpu.VMEM((2, CH), jnp.int32),
            pltpu.VMEM((2, CH), jnp.int32),
            pltpu.VMEM((CH, hc), jnp.float32),
            pltpu.VMEM((CH, hc), jnp.float32),
            pltpu.VMEM_SHARED((NPAD, hc), jnp.float32),
        ]
        + [pltpu.SemaphoreType.DMA] * 6,
    )
    return f(y0, y1, sd_pad)


def _mp_edge_body(hc, y0, y1, sd_hbm, z_hbm, s0_out, s1_out,
                  sd0, sd1, r0, r1, acc, *sems):
    c = lax.axis_index("c")
    s = lax.axis_index("s")

    @pl.when(c == 0)
    def _():
        pltpu.sync_copy(y0.at[pl.ds(s * RPT, RPT)], acc.at[pl.ds(s * RPT, RPT)])
        _mp_common(CPT_E, NC * s * CPT_E, y0, sd_hbm, (sd0, sd1), (r0, r1),
                   acc, sems, stride=NC)

    @pl.when(c == 1)
    def _():
        pltpu.sync_copy(z_hbm.at[pl.ds(s * RPT, RPT)], acc.at[pl.ds(s * RPT, RPT)])
        _mp_common(CPT_E, NC * s * CPT_E + 1, y1, sd_hbm, (sd0, sd1), (r0, r1),
                   acc, sems, stride=NC)

    _mp_writeback(c, s, acc, s0_out, s1_out)


def _mp_edge_call(hc, y0, y1, sd_pad, z):
    f = pl.kernel(
        functools.partial(_mp_edge_body, hc),
        out_type=[jax.ShapeDtypeStruct((NPAD, hc), jnp.float32)] * 2,
        mesh=_mesh,
        scratch_types=[
            pltpu.VMEM((2, CH), jnp.int32),
            pltpu.VMEM((2, CH), jnp.int32),
            pltpu.VMEM((CH, hc), jnp.float32),
            pltpu.VMEM((CH, hc), jnp.float32),
            pltpu.VMEM_SHARED((NPAD, hc), jnp.float32),
        ]
        + [pltpu.SemaphoreType.DMA] * 6,
    )
    return f(y0, y1, sd_pad, z)


def _dis(deg_ref):
    return lax.rsqrt(deg_ref[...] + 1.0)


def _lin1_body(x_ref, w_ref, deg_ref, y0_ref, y1_ref):
    dis = _dis(deg_ref)
    y = jnp.dot(x_ref[...], w_ref[...], preferred_element_type=jnp.float32) * dis
    y0_ref[...] = y[:, : H2 // 2]
    y1_ref[...] = y[:, H2 // 2 :]


def _lin1_call(x_pad, w0, deg2):
    return pl.pallas_call(
        _lin1_body,
        grid=(GRID2,),
        in_specs=[
            pl.BlockSpec((RB2, D_IN), lambda i: (i, 0)),
            pl.BlockSpec((D_IN, H2), lambda i: (0, 0)),
            pl.BlockSpec((RB2, 1), lambda i: (i, 0)),
        ],
        out_specs=[
            pl.BlockSpec((RB2, H2 // 2), lambda i: (i, 0)),
            pl.BlockSpec((RB2, H2 // 2), lambda i: (i, 0)),
        ],
        out_shape=[jax.ShapeDtypeStruct((NPAD, H2 // 2), jnp.float32)] * 2,
    )(x_pad, w0, deg2)


def _mid_body(split_out, s0_ref, s1_ref, deg_ref, w_ref, b_ref, *out_refs):
    dis = _dis(deg_ref)
    h = jnp.concatenate([s0_ref[...], s1_ref[...]], axis=1)
    h = jax.nn.relu(dis * h + b_ref[...])
    y = jnp.dot(h, w_ref[...], preferred_element_type=jnp.float32) * dis
    row = pl.program_id(0) * RB2 + jax.lax.broadcasted_iota(
        jnp.int32, (RB2, 1), 0
    )
    y = jnp.where(row < N, y, 0.0)
    if split_out:
        hh = w_ref.shape[1] // 2
        out_refs[0][...] = y[:, :hh]
        out_refs[1][...] = y[:, hh:]
    else:
        out_refs[0][...] = y
        out_refs[1][...] = y


def _mid_call(s0, s1, deg2, w, b2d, split_out=True):
    hin = w.shape[0]
    hout = w.shape[1]
    if split_out:
        out_specs = [
            pl.BlockSpec((RB2, hout // 2), lambda i: (i, 0)),
            pl.BlockSpec((RB2, hout // 2), lambda i: (i, 0)),
        ]
        out_shape = [jax.ShapeDtypeStruct((NPAD, hout // 2), jnp.float32)] * 2
    else:
        out_specs = [
            pl.BlockSpec((RB2, hout), lambda i: (i, 0)),
            pl.BlockSpec((RB2, hout), lambda i: (i, 0)),
        ]
        out_shape = [jax.ShapeDtypeStruct((NPAD, hout), jnp.float32)] * 2
    return pl.pallas_call(
        functools.partial(_mid_body, split_out),
        grid=(GRID2,),
        in_specs=[
            pl.BlockSpec((RB2, hin // 2), lambda i: (i, 0)),
            pl.BlockSpec((RB2, hin // 2), lambda i: (i, 0)),
            pl.BlockSpec((RB2, 1), lambda i: (i, 0)),
            pl.BlockSpec((hin, hout), lambda i: (0, 0)),
            pl.BlockSpec((1, hin), lambda i: (0, 0)),
        ],
        out_specs=out_specs,
        out_shape=out_shape,
    )(s0, s1, deg2, w, b2d)


def _fin_body(s0_ref, s1_ref, deg_ref, b2_ref, wp_ref, bp_ref, out_ref):
    dis = _dis(deg_ref)
    h = s0_ref[...] + s1_ref[...]
    h = jax.nn.relu(dis * h + b2_ref[...])
    p = jax.nn.relu(
        jnp.dot(h, wp_ref[...], preferred_element_type=jnp.float32) + bp_ref[...]
    )
    nrm = jnp.sqrt(jnp.sum(p * p, axis=1, keepdims=True))
    out_ref[...] = p / jnp.maximum(nrm, 1e-12)


def _fin_call(s0, s1, deg2, b2d, wp, bp2d):
    return pl.pallas_call(
        _fin_body,
        grid=(GRID,),
        in_specs=[
            pl.BlockSpec((RB, HID), lambda i: (i, 0)),
            pl.BlockSpec((RB, HID), lambda i: (i, 0)),
            pl.BlockSpec((RB, 1), lambda i: (i, 0)),
            pl.BlockSpec((1, HID), lambda i: (0, 0)),
            pl.BlockSpec((HID, PROJ), lambda i: (0, 0)),
            pl.BlockSpec((1, PROJ), lambda i: (0, 0)),
        ],
        out_specs=pl.BlockSpec((RB, PROJ), lambda i: (i, 0)),
        out_shape=jax.ShapeDtypeStruct((N, PROJ), jnp.float32),
    )(s0, s1, deg2, b2d, wp, bp2d)


def kernel(x, edge_index1, W0, b0, W1, b1, W2, b2, Wp, bp):
    pad = E_PAD - E
    ar = jnp.arange(pad, dtype=jnp.int32)
    src_pad = jnp.concatenate([edge_index1[0], DUMMY + ar % (NPAD - N)])
    dst_pad = jnp.concatenate([edge_index1[1], ar % NPAD])
    sd_pad = jnp.stack(
        [src_pad.reshape(PADC, CH), dst_pad.reshape(PADC, CH)], axis=1
    )
    dd_pad = jnp.concatenate(
        [edge_index1[1], DUMMY + ar % (NPAD - N)]
    ).reshape(PADC, 1, CH)
    z1 = jnp.zeros((NPAD,), jnp.float32)
    z2 = jnp.zeros((NPAD, H2 // 2), jnp.float32)
    x_pad = jnp.concatenate([x, jnp.zeros((NPAD - N, D_IN), jnp.float32)])

    deg = _deg_call(dd_pad, z1)
    deg2 = deg[:, None]

    y0a, y0b = _lin1_call(x_pad, W0, deg2)
    s1a, s1b = _mp_call(H2 // 2, y0a, y0b, sd_pad)

    y1a, y1b = _mid_call(s1a, s1b, deg2, W1, b0[None, :])
    s2a, s2b = _mp_call(H2 // 2, y1a, y1b, sd_pad)

    y2a, y2b = _mid_call(s2a, s2b, deg2, W2, b1[None, :], split_out=False)
    s3a, s3b = _mp_edge_call(HID, y2a, y2b, sd_pad, z2)

    return _fin_call(s3a, s3b, deg2, b2[None, :], Wp, bp[None, :])
